# baseline (speedup 1.0000x reference)
import functools

import jax
import jax.numpy as jnp
from jax import lax
from jax.experimental import pallas as pl
from jax.experimental.pallas import tpu as pltpu
from jax.experimental.pallas import tpu_sc as plsc

NC = 2
NS = 16
NW = NC * NS

EK = 128
NSEG = 2


def _chunk_geometry(E):
    ept = E // NW
    assert E % NW == 0
    n_chunks = -(-ept // EK)
    n_chunks = -(-n_chunks // (2 * NSEG)) * (2 * NSEG)
    return ept, n_chunks


def _row_partition(NP):
    base = (NP // NS) // 8 * 8
    tail = NP - base * NS
    assert tail % 8 == 0 and base % 8 == 0
    return base, tail


def _make_deg_pass(E, NP, H):
    _, n_chunks = _chunk_geometry(E)
    rows_per_tile, row_tail = _row_partition(NP)
    mesh = plsc.VectorSubcoreMesh(core_axis_name="c", subcore_axis_name="s")

    @functools.partial(
        pl.kernel,
        out_type=jax.ShapeDtypeStruct((NC, NP, H), jnp.float32),
        mesh=mesh,
        scratch_types=[
            pltpu.VMEM((n_chunks, EK), jnp.int32),
            pltpu.VMEM((EK, H), jnp.float32),
            pltpu.VMEM_SHARED((NP, H), jnp.float32),
        ],
    )
    def deg_pass(dst_hbm, ones_hbm, z_hbm, degp_hbm, dstv, onesv, dacc):
        c = lax.axis_index("c")
        s = lax.axis_index("s")
        wid = c * NS + s
        pltpu.sync_copy(ones_hbm, onesv)
        pltpu.sync_copy(dst_hbm.at[wid], dstv)
        r0 = s * rows_per_tile
        rt = NS * rows_per_tile
        pltpu.sync_copy(z_hbm.at[pl.ds(r0, rows_per_tile)],
                        dacc.at[pl.ds(r0, rows_per_tile)])

        if row_tail:
            @pl.when(s == 0)
            def _():
                pltpu.sync_copy(z_hbm.at[pl.ds(rt, row_tail)],
                                dacc.at[pl.ds(rt, row_tail)])

        plsc.subcore_barrier()

        def body(i, _):
            pltpu.sync_copy(onesv, dacc.at[dstv.at[i]], add=True)
            return ()

        lax.fori_loop(0, n_chunks, body, ())
        plsc.subcore_barrier()
        pltpu.sync_copy(dacc.at[pl.ds(r0, rows_per_tile)],
                        degp_hbm.at[c, pl.ds(r0, rows_per_tile)])

        if row_tail:
            @pl.when(s == 0)
            def _():
                pltpu.sync_copy(dacc.at[pl.ds(rt, row_tail)],
                                degp_hbm.at[c, pl.ds(rt, row_tail)])

    return deg_pass


def _make_edge_pass(E, NP, H):
    _, n_chunks = _chunk_geometry(E)
    ek = EK
    assert n_chunks % 2 == 0
    rows_per_tile, row_tail = _row_partition(NP)
    mesh = plsc.VectorSubcoreMesh(core_axis_name="c", subcore_axis_name="s")

    @functools.partial(
        pl.kernel,
        out_type=jax.ShapeDtypeStruct((NC, NP, H), jnp.float32),
        mesh=mesh,
        scratch_types=[
            [pltpu.VMEM((ek,), jnp.int32)] * 2,
            [pltpu.VMEM((ek,), jnp.int32)] * 2,
            [pltpu.VMEM((ek, H), jnp.float32)] * 2,
            pltpu.VMEM_SHARED((NP, H), jnp.float32),
            [pltpu.SemaphoreType.DMA] * 2,
            [pltpu.SemaphoreType.DMA] * 2,
        ],
    )
    def edge_pass(src_hbm, dst_hbm, q_hbm, z_hbm, out_hbm,
                  sidx, didx, rows, acc, g, sc):
        c = lax.axis_index("c")
        s = lax.axis_index("s")
        wid = c * NS + s
        r0 = s * rows_per_tile
        rt = NS * rows_per_tile
        pltpu.sync_copy(z_hbm.at[pl.ds(r0, rows_per_tile)],
                        acc.at[pl.ds(r0, rows_per_tile)])

        if row_tail:
            @pl.when(s == 0)
            def _():
                pltpu.sync_copy(z_hbm.at[pl.ds(rt, row_tail)],
                                acc.at[pl.ds(rt, row_tail)])

        plsc.subcore_barrier()

        e_base = wid * (n_chunks * ek)

        def chunk_slice(cc):
            return pl.ds(e_base + cc * ek, ek)

        pltpu.sync_copy(src_hbm.at[chunk_slice(0)], sidx[0])
        pltpu.sync_copy(dst_hbm.at[chunk_slice(0)], didx[0])
        pltpu.async_copy(q_hbm.at[sidx[0]], rows[0], g[0])

        def body(k, _):
            for j in range(2):
                cc = 2 * k + j
                j1 = (j + 1) % 2

                pltpu.make_async_copy(
                    q_hbm.at[sidx[j]], rows[j], g[j]).wait()
                d = pltpu.async_copy(
                    rows[j], acc.at[didx[j]], sc[j], add=True)

                @pl.when(cc + 1 < n_chunks)
                def _():
                    pltpu.sync_copy(src_hbm.at[chunk_slice(cc + 1)],
                                    sidx[j1])
                    pltpu.sync_copy(dst_hbm.at[chunk_slice(cc + 1)],
                                    didx[j1])
                    pltpu.async_copy(q_hbm.at[sidx[j1]], rows[j1], g[j1])

                d.wait()

            return ()

        lax.fori_loop(0, n_chunks // 2, body, ())
        plsc.subcore_barrier()
        pltpu.sync_copy(acc.at[pl.ds(r0, rows_per_tile)],
                        out_hbm.at[c, pl.ds(r0, rows_per_tile)])

        if row_tail:
            @pl.when(s == 0)
            def _():
                pltpu.sync_copy(acc.at[pl.ds(rt, row_tail)],
                                out_hbm.at[c, pl.ds(rt, row_tail)])

    return edge_pass


def _tc1_body(degp_ref, x_ref, w1_ref, out_ref):
    dinv = lax.rsqrt(degp_ref[0] + degp_ref[1] + 1.0)
    p = jnp.dot(x_ref[...], w1_ref[...], preferred_element_type=jnp.float32)
    out_ref[...] = p * dinv


def _tc2_body(degp_ref, aggp_ref, q_ref, w2_ref, b1_ref, out_ref):
    dinv = lax.rsqrt(degp_ref[0] + degp_ref[1] + 1.0)
    h1 = jnp.maximum(
        dinv * (aggp_ref[0] + aggp_ref[1] + q_ref[...]) + b1_ref[...], 0.0)
    out_ref[...] = jnp.dot(
        h1, w2_ref[...], preferred_element_type=jnp.float32) * dinv


def _tc3_body(nblk, B, degp_ref, aggp_ref, q_ref, b2_ref, batch_ref,
              wlin_ref, blin_ref, out_ref, sums_ref, cnts_ref):
    m = pl.program_id(0)

    @pl.when(m == 0)
    def _():
        sums_ref[...] = jnp.zeros_like(sums_ref)
        cnts_ref[...] = jnp.zeros_like(cnts_ref)

    dinv = lax.rsqrt(degp_ref[0] + degp_ref[1] + 1.0)
    z = dinv * (aggp_ref[0] + aggp_ref[1] + q_ref[...]) + b2_ref[...]
    h = jnp.maximum(z, 0.0)
    mb = h.shape[0]
    bids = batch_ref[0, 0, :]
    seg = lax.broadcasted_iota(jnp.int32, (B, mb), 0)
    msk = (bids[None, :] == seg).astype(jnp.float32)
    sums_ref[...] += jnp.dot(msk, h, preferred_element_type=jnp.float32)
    cnts_ref[...] += jnp.broadcast_to(
        jnp.sum(msk, axis=1, keepdims=True), cnts_ref.shape)

    @pl.when(m == nblk - 1)
    def _():
        pooled = sums_ref[...] / jnp.maximum(cnts_ref[...], 1.0)
        out_ref[...] = jnp.dot(
            pooled, wlin_ref[...],
            preferred_element_type=jnp.float32) + blin_ref[...]


def kernel(x, edge_index, batch, W1, b1, W2, b2, Wlin, blin):
    N, D = x.shape
    H = W1.shape[1]
    C = Wlin.shape[1]
    E = edge_index.shape[1]
    B = 64

    MB = 1000
    nblk = N // MB
    assert N % MB == 0

    ept, n_chunks = _chunk_geometry(E)
    pad = n_chunks * EK - ept
    NP = N + max(pad, 8)
    assert NP % 8 == 0
    MBP = NP // 10
    nblkp = NP // MBP
    assert NP % MBP == 0 and MBP % 8 == 0

    sink = N + (jnp.arange(pad, dtype=jnp.int32) % max(pad, 1))
    src = edge_index[0].reshape(NW, ept)
    dst = edge_index[1].reshape(NW, ept)
    src = jnp.concatenate(
        [src, jnp.broadcast_to(sink, (NW, pad))], axis=1)
    dst = jnp.concatenate(
        [dst, jnp.broadcast_to(sink, (NW, pad))], axis=1)
    src_flat = src.reshape(NW * n_chunks * EK)
    dst_flat = dst.reshape(NW * n_chunks * EK)
    dst = dst.reshape(NW, n_chunks, EK)

    zerosH = jnp.zeros((NP, H), jnp.float32)
    onesH = jnp.ones((EK, H), jnp.float32)
    batch3 = batch.reshape(nblk, 1, MB)
    b1r = b1.reshape(1, H)
    b2r = b2.reshape(1, H)
    blinr = blin.reshape(1, C)

    deg_pass = _make_deg_pass(E, NP, H)
    edge_pass = _make_edge_pass(E, NP, H)

    degp = deg_pass(dst, onesH, zerosH)

    q1 = pl.pallas_call(
        _tc1_body,
        grid=(nblkp,),
        in_specs=[
            pl.BlockSpec((NC, MBP, H), lambda m: (0, m, 0)),
            pl.BlockSpec((MBP, D), lambda m: (m, 0)),
            pl.BlockSpec((D, H), lambda m: (0, 0)),
        ],
        out_specs=pl.BlockSpec((MBP, H), lambda m: (m, 0)),
        out_shape=jax.ShapeDtypeStruct((NP, H), jnp.float32),
    )(degp, x, W1)

    aggp1 = edge_pass(src_flat, dst_flat, q1, zerosH)

    q2 = pl.pallas_call(
        _tc2_body,
        grid=(nblkp,),
        in_specs=[
            pl.BlockSpec((NC, MBP, H), lambda m: (0, m, 0)),
            pl.BlockSpec((NC, MBP, H), lambda m: (0, m, 0)),
            pl.BlockSpec((MBP, H), lambda m: (m, 0)),
            pl.BlockSpec((H, H), lambda m: (0, 0)),
            pl.BlockSpec((1, H), lambda m: (0, 0)),
        ],
        out_specs=pl.BlockSpec((MBP, H), lambda m: (m, 0)),
        out_shape=jax.ShapeDtypeStruct((NP, H), jnp.float32),
    )(degp, aggp1, q1, W2, b1r)

    aggp2 = edge_pass(src_flat, dst_flat, q2, zerosH)

    logits = pl.pallas_call(
        functools.partial(_tc3_body, nblk, B),
        grid=(nblk,),
        in_specs=[
            pl.BlockSpec((NC, MB, H), lambda m: (0, m, 0)),
            pl.BlockSpec((NC, MB, H), lambda m: (0, m, 0)),
            pl.BlockSpec((MB, H), lambda m: (m, 0)),
            pl.BlockSpec((1, H), lambda m: (0, 0)),
            pl.BlockSpec((1, 1, MB), lambda m: (m, 0, 0)),
            pl.BlockSpec((H, C), lambda m: (0, 0)),
            pl.BlockSpec((1, C), lambda m: (0, 0)),
        ],
        out_specs=pl.BlockSpec((B, C), lambda m: (0, 0)),
        out_shape=jax.ShapeDtypeStruct((B, C), jnp.float32),
        scratch_shapes=[
            pltpu.VMEM((B, H), jnp.float32),
            pltpu.VMEM((B, H), jnp.float32),
        ],
    )(degp, aggp2, q2, b2r, batch3, Wlin, blinr)

    return logits


# speedup vs baseline: 1.4713x; 1.4713x over previous
import functools

import jax
import jax.numpy as jnp
from jax import lax
from jax.experimental import pallas as pl
from jax.experimental.pallas import tpu as pltpu
from jax.experimental.pallas import tpu_sc as plsc

NC = 2
NS = 16
NW = NC * NS

EK = 128
NSEG = 2


def _chunk_geometry(E):
    ept = E // NW
    assert E % NW == 0
    n_chunks = -(-ept // EK)
    n_chunks = -(-n_chunks // (2 * NSEG)) * (2 * NSEG)
    return ept, n_chunks


def _row_partition(NP):
    base = (NP // NS) // 8 * 8
    tail = NP - base * NS
    assert tail % 8 == 0 and base % 8 == 0
    return base, tail


def _make_deg_pass(E, NP, H):
    _, n_chunks = _chunk_geometry(E)
    rows_per_tile, row_tail = _row_partition(NP)
    mesh = plsc.VectorSubcoreMesh(core_axis_name="c", subcore_axis_name="s")

    @functools.partial(
        pl.kernel,
        out_type=jax.ShapeDtypeStruct((NC, NP, H), jnp.float32),
        mesh=mesh,
        scratch_types=[
            pltpu.VMEM((n_chunks, EK), jnp.int32),
            pltpu.VMEM((EK, H), jnp.float32),
            pltpu.VMEM_SHARED((NP, H), jnp.float32),
        ],
    )
    def deg_pass(dst_hbm, ones_hbm, z_hbm, degp_hbm, dstv, onesv, dacc):
        c = lax.axis_index("c")
        s = lax.axis_index("s")
        wid = c * NS + s
        pltpu.sync_copy(ones_hbm, onesv)
        pltpu.sync_copy(dst_hbm.at[wid], dstv)
        r0 = s * rows_per_tile
        rt = NS * rows_per_tile
        pltpu.sync_copy(z_hbm.at[pl.ds(r0, rows_per_tile)],
                        dacc.at[pl.ds(r0, rows_per_tile)])

        if row_tail:
            @pl.when(s == 0)
            def _():
                pltpu.sync_copy(z_hbm.at[pl.ds(rt, row_tail)],
                                dacc.at[pl.ds(rt, row_tail)])

        plsc.subcore_barrier()

        def body(i, _):
            pltpu.sync_copy(onesv, dacc.at[dstv.at[i]], add=True)
            return ()

        lax.fori_loop(0, n_chunks, body, ())
        plsc.subcore_barrier()
        pltpu.sync_copy(dacc.at[pl.ds(r0, rows_per_tile)],
                        degp_hbm.at[c, pl.ds(r0, rows_per_tile)])

        if row_tail:
            @pl.when(s == 0)
            def _():
                pltpu.sync_copy(dacc.at[pl.ds(rt, row_tail)],
                                degp_hbm.at[c, pl.ds(rt, row_tail)])

    return deg_pass


def _make_edge_pass(E, NP, H):
    _, n_chunks = _chunk_geometry(E)
    ek = EK
    assert n_chunks % 2 == 0
    rows_per_tile, row_tail = _row_partition(NP)
    mesh = plsc.VectorSubcoreMesh(core_axis_name="c", subcore_axis_name="s")

    segc = n_chunks // NSEG
    assert segc % 2 == 0

    @functools.partial(
        pl.kernel,
        out_type=jax.ShapeDtypeStruct((NC, NP, H), jnp.float32),
        mesh=mesh,
        scratch_types=[
            pltpu.VMEM((segc, ek), jnp.int32),
            pltpu.VMEM((segc, ek), jnp.int32),
            [pltpu.VMEM((ek, H), jnp.float32)] * 2,
            pltpu.VMEM_SHARED((NP, H), jnp.float32),
            [pltpu.SemaphoreType.DMA] * 2,
        ],
    )
    def edge_pass(src_hbm, dst_hbm, q_hbm, z_hbm, out_hbm,
                  sidx, didx, rows, acc, g):
        c = lax.axis_index("c")
        s = lax.axis_index("s")
        wid = c * NS + s
        r0 = s * rows_per_tile
        rt = NS * rows_per_tile
        pltpu.sync_copy(z_hbm.at[pl.ds(r0, rows_per_tile)],
                        acc.at[pl.ds(r0, rows_per_tile)])

        if row_tail:
            @pl.when(s == 0)
            def _():
                pltpu.sync_copy(z_hbm.at[pl.ds(rt, row_tail)],
                                acc.at[pl.ds(rt, row_tail)])

        plsc.subcore_barrier()

        for seg in range(NSEG):
            cbase = seg * segc
            pltpu.sync_copy(src_hbm.at[wid, pl.ds(cbase, segc)], sidx)
            pltpu.sync_copy(dst_hbm.at[wid, pl.ds(cbase, segc)], didx)
            pltpu.async_copy(q_hbm.at[sidx.at[0]], rows[0], g[0])

            def body(k, _):
                for j in range(2):
                    cc = 2 * k + j
                    j1 = (j + 1) % 2

                    @pl.when(cc + 1 < segc)
                    def _():
                        pltpu.async_copy(
                            q_hbm.at[sidx.at[cc + 1]], rows[j1], g[j1])

                    pltpu.make_async_copy(
                        q_hbm.at[sidx.at[cc]], rows[j], g[j]).wait()
                    pltpu.sync_copy(rows[j], acc.at[didx.at[cc]], add=True)

                return ()

            lax.fori_loop(0, segc // 2, body, ())

        plsc.subcore_barrier()
        pltpu.sync_copy(acc.at[pl.ds(r0, rows_per_tile)],
                        out_hbm.at[c, pl.ds(r0, rows_per_tile)])

        if row_tail:
            @pl.when(s == 0)
            def _():
                pltpu.sync_copy(acc.at[pl.ds(rt, row_tail)],
                                out_hbm.at[c, pl.ds(rt, row_tail)])

    return edge_pass


def _tc1_body(degp_ref, x_ref, w1_ref, out_ref):
    dinv = lax.rsqrt(degp_ref[0] + degp_ref[1] + 1.0)
    p = jnp.dot(x_ref[...], w1_ref[...], preferred_element_type=jnp.float32)
    out_ref[...] = p * dinv


def _tc2_body(degp_ref, aggp_ref, q_ref, w2_ref, b1_ref, out_ref):
    dinv = lax.rsqrt(degp_ref[0] + degp_ref[1] + 1.0)
    h1 = jnp.maximum(
        dinv * (aggp_ref[0] + aggp_ref[1] + q_ref[...]) + b1_ref[...], 0.0)
    out_ref[...] = jnp.dot(
        h1, w2_ref[...], preferred_element_type=jnp.float32) * dinv


def _tc3_body(nblk, B, degp_ref, aggp_ref, q_ref, b2_ref, batch_ref,
              wlin_ref, blin_ref, out_ref, sums_ref, cnts_ref):
    m = pl.program_id(0)

    @pl.when(m == 0)
    def _():
        sums_ref[...] = jnp.zeros_like(sums_ref)
        cnts_ref[...] = jnp.zeros_like(cnts_ref)

    dinv = lax.rsqrt(degp_ref[0] + degp_ref[1] + 1.0)
    z = dinv * (aggp_ref[0] + aggp_ref[1] + q_ref[...]) + b2_ref[...]
    h = jnp.maximum(z, 0.0)
    mb = h.shape[0]
    bids = batch_ref[0, 0, :]
    seg = lax.broadcasted_iota(jnp.int32, (B, mb), 0)
    msk = (bids[None, :] == seg).astype(jnp.float32)
    sums_ref[...] += jnp.dot(msk, h, preferred_element_type=jnp.float32)
    cnts_ref[...] += jnp.broadcast_to(
        jnp.sum(msk, axis=1, keepdims=True), cnts_ref.shape)

    @pl.when(m == nblk - 1)
    def _():
        pooled = sums_ref[...] / jnp.maximum(cnts_ref[...], 1.0)
        out_ref[...] = jnp.dot(
            pooled, wlin_ref[...],
            preferred_element_type=jnp.float32) + blin_ref[...]


def kernel(x, edge_index, batch, W1, b1, W2, b2, Wlin, blin):
    N, D = x.shape
    H = W1.shape[1]
    C = Wlin.shape[1]
    E = edge_index.shape[1]
    B = 64

    MB = 1000
    nblk = N // MB
    assert N % MB == 0

    ept, n_chunks = _chunk_geometry(E)
    pad = n_chunks * EK - ept
    NP = N + max(pad, 8)
    assert NP % 8 == 0
    MBP = NP // 10
    nblkp = NP // MBP
    assert NP % MBP == 0 and MBP % 8 == 0

    sink = N + (jnp.arange(pad, dtype=jnp.int32) % max(pad, 1))
    src = edge_index[0].reshape(NW, ept)
    dst = edge_index[1].reshape(NW, ept)
    src = jnp.concatenate(
        [src, jnp.broadcast_to(sink, (NW, pad))], axis=1)
    dst = jnp.concatenate(
        [dst, jnp.broadcast_to(sink, (NW, pad))], axis=1)
    src = src.reshape(NW, n_chunks, EK)
    dst = dst.reshape(NW, n_chunks, EK)

    zerosH = jnp.zeros((NP, H), jnp.float32)
    onesH = jnp.ones((EK, H), jnp.float32)
    batch3 = batch.reshape(nblk, 1, MB)
    b1r = b1.reshape(1, H)
    b2r = b2.reshape(1, H)
    blinr = blin.reshape(1, C)

    deg_pass = _make_deg_pass(E, NP, H)
    edge_pass = _make_edge_pass(E, NP, H)

    degp = deg_pass(dst, onesH, zerosH)

    q1 = pl.pallas_call(
        _tc1_body,
        grid=(nblkp,),
        in_specs=[
            pl.BlockSpec((NC, MBP, H), lambda m: (0, m, 0)),
            pl.BlockSpec((MBP, D), lambda m: (m, 0)),
            pl.BlockSpec((D, H), lambda m: (0, 0)),
        ],
        out_specs=pl.BlockSpec((MBP, H), lambda m: (m, 0)),
        out_shape=jax.ShapeDtypeStruct((NP, H), jnp.float32),
    )(degp, x, W1)

    aggp1 = edge_pass(src, dst, q1, zerosH)

    q2 = pl.pallas_call(
        _tc2_body,
        grid=(nblkp,),
        in_specs=[
            pl.BlockSpec((NC, MBP, H), lambda m: (0, m, 0)),
            pl.BlockSpec((NC, MBP, H), lambda m: (0, m, 0)),
            pl.BlockSpec((MBP, H), lambda m: (m, 0)),
            pl.BlockSpec((H, H), lambda m: (0, 0)),
            pl.BlockSpec((1, H), lambda m: (0, 0)),
        ],
        out_specs=pl.BlockSpec((MBP, H), lambda m: (m, 0)),
        out_shape=jax.ShapeDtypeStruct((NP, H), jnp.float32),
    )(degp, aggp1, q1, W2, b1r)

    aggp2 = edge_pass(src, dst, q2, zerosH)

    logits = pl.pallas_call(
        functools.partial(_tc3_body, nblk, B),
        grid=(nblk,),
        in_specs=[
            pl.BlockSpec((NC, MB, H), lambda m: (0, m, 0)),
            pl.BlockSpec((NC, MB, H), lambda m: (0, m, 0)),
            pl.BlockSpec((MB, H), lambda m: (m, 0)),
            pl.BlockSpec((1, H), lambda m: (0, 0)),
            pl.BlockSpec((1, 1, MB), lambda m: (m, 0, 0)),
            pl.BlockSpec((H, C), lambda m: (0, 0)),
            pl.BlockSpec((1, C), lambda m: (0, 0)),
        ],
        out_specs=pl.BlockSpec((B, C), lambda m: (0, 0)),
        out_shape=jax.ShapeDtypeStruct((B, C), jnp.float32),
        scratch_shapes=[
            pltpu.VMEM((B, H), jnp.float32),
            pltpu.VMEM((B, H), jnp.float32),
        ],
    )(degp, aggp2, q2, b2r, batch3, Wlin, blinr)

    return logits
